# baseline (speedup 1.0000x reference)
import functools

import jax
import jax.numpy as jnp
from jax import lax
from jax.experimental import pallas as pl
from jax.experimental.pallas import tpu as pltpu
from jax.experimental.pallas import tpu_sc as plsc

N = 10000
E = 320000
HID = 128
CHUNK = 80
NW = 32
EDGES_PER_W = E // NW
CHUNKS_PER_W = EDGES_PER_W // CHUNK
DEG_LANES = 16
ROWS_A = 640
ROWS_B = N - 15 * ROWS_A

_sc_mesh = functools.partial(
    plsc.VectorSubcoreMesh, core_axis_name="c", subcore_axis_name="s")


def _init_rows(src_hbm_like, acc, s):
    @pl.when(s < 15)
    def _():
        pltpu.sync_copy(src_hbm_like.at[pl.ds(s * ROWS_A, ROWS_A)],
                        acc.at[pl.ds(s * ROWS_A, ROWS_A)])

    @pl.when(s == 15)
    def _():
        pltpu.sync_copy(src_hbm_like.at[pl.ds(15 * ROWS_A, ROWS_B)],
                        acc.at[pl.ds(15 * ROWS_A, ROWS_B)])


def _drain_rows(acc, out_hbm, c, s):
    @pl.when(s < 15)
    def _():
        pltpu.sync_copy(acc.at[pl.ds(s * ROWS_A, ROWS_A)],
                        out_hbm.at[c, pl.ds(s * ROWS_A, ROWS_A)])

    @pl.when(s == 15)
    def _():
        pltpu.sync_copy(acc.at[pl.ds(15 * ROWS_A, ROWS_B)],
                        out_hbm.at[c, pl.ds(15 * ROWS_A, ROWS_B)])


def _deg_kernel(dst_hbm, ones_hbm, ones_c_hbm, out_hbm, dst_slab, ones_v, acc,
                sem):
    c = lax.axis_index("c")
    s = lax.axis_index("s")
    w = c * 16 + s
    _init_rows(ones_hbm, acc, s)
    pltpu.sync_copy(ones_c_hbm, ones_v)
    pltpu.sync_copy(dst_hbm.at[w], dst_slab)
    plsc.subcore_barrier()

    def body(k, carry):
        for b in range(5):
            pltpu.async_copy(ones_v, acc.at[dst_slab.at[5 * k + b]], sem,
                             add=True)
        for b in range(5):
            pltpu.make_async_copy(ones_v, acc.at[dst_slab.at[0]], sem).wait()
        return carry

    lax.fori_loop(0, CHUNKS_PER_W // 5, body, 0)
    plsc.subcore_barrier()
    _drain_rows(acc, out_hbm, c, s)


def _seg_kernel(hs_hbm, src_hbm, dst_hbm, out_hbm, src_v, dst_slab,
                rows_a, rows_b, acc, gsem_a, gsem_b):
    c = lax.axis_index("c")
    s = lax.axis_index("s")
    w = c * 16 + s
    _init_rows(hs_hbm, acc, s)
    pltpu.sync_copy(src_hbm.at[pl.ds(w * EDGES_PER_W, EDGES_PER_W)], src_v)
    pltpu.sync_copy(dst_hbm.at[w], dst_slab)
    plsc.subcore_barrier()

    def gather(j, buf, sem):
        return pltpu.async_copy(
            hs_hbm.at[src_v.at[pl.ds(j * CHUNK, CHUNK)]], buf, sem)

    gather(0, rows_a, gsem_a)
    gather(1, rows_b, gsem_b)

    def drain(buf, sem):
        pltpu.make_async_copy(
            hs_hbm.at[src_v.at[pl.ds(0, CHUNK)]], buf, sem).wait()

    def body(k, carry):
        drain(rows_a, gsem_a)
        pltpu.sync_copy(rows_a, acc.at[dst_slab.at[2 * k]], add=True)
        gather(2 * k + 2, rows_a, gsem_a)
        drain(rows_b, gsem_b)
        pltpu.sync_copy(rows_b, acc.at[dst_slab.at[2 * k + 1]], add=True)
        gather(2 * k + 3, rows_b, gsem_b)
        return carry

    lax.fori_loop(0, (CHUNKS_PER_W - 3) // 2, body, 0)
    drain(rows_a, gsem_a)
    pltpu.sync_copy(rows_a, acc.at[dst_slab.at[CHUNKS_PER_W - 3]], add=True)
    gather(CHUNKS_PER_W - 1, rows_a, gsem_a)
    drain(rows_b, gsem_b)
    pltpu.sync_copy(rows_b, acc.at[dst_slab.at[CHUNKS_PER_W - 2]], add=True)
    drain(rows_a, gsem_a)
    pltpu.sync_copy(rows_a, acc.at[dst_slab.at[CHUNKS_PER_W - 1]], add=True)
    plsc.subcore_barrier()
    _drain_rows(acc, out_hbm, c, s)


def _sc_deg(dst3d, ones, ones_c):
    k = pl.kernel(
        _deg_kernel,
        out_type=jax.ShapeDtypeStruct((2, N, DEG_LANES), jnp.float32),
        mesh=_sc_mesh(),
        scratch_types=[
            pltpu.VMEM((CHUNKS_PER_W, CHUNK), jnp.int32),
            pltpu.VMEM((CHUNK, DEG_LANES), jnp.float32),
            pltpu.VMEM_SHARED((N, DEG_LANES), jnp.float32),
            pltpu.SemaphoreType.DMA,
        ],
    )
    return k(dst3d, ones, ones_c)


def _sc_seg(hs, src1d, dst3d):
    k = pl.kernel(
        _seg_kernel,
        out_type=jax.ShapeDtypeStruct((2, N, HID), jnp.float32),
        mesh=_sc_mesh(),
        scratch_types=[
            pltpu.VMEM((EDGES_PER_W,), jnp.int32),
            pltpu.VMEM((CHUNKS_PER_W, CHUNK), jnp.int32),
            pltpu.VMEM((CHUNK, HID), jnp.float32),
            pltpu.VMEM((CHUNK, HID), jnp.float32),
            pltpu.VMEM_SHARED((N, HID), jnp.float32),
            pltpu.SemaphoreType.DMA,
            pltpu.SemaphoreType.DMA,
        ],
    )
    return k(hs, src1d, dst3d)


BLK = 1000


def _u1_body(x_ref, e_ref, w_ref, o_ref):
    h = jnp.concatenate([x_ref[...], e_ref[...]], axis=1)
    o_ref[...] = lax.dot_general(
        h, w_ref[...], (((1,), (0,)), ((), ())),
        preferred_element_type=jnp.float32)


def _tc_u1(x, emb_blk, W1):
    return pl.pallas_call(
        _u1_body,
        grid=(N // BLK,),
        in_specs=[
            pl.BlockSpec((BLK, 124), lambda i: (i, 0)),
            pl.BlockSpec((BLK, 4), lambda i: (0, 0)),
            pl.BlockSpec((HID, HID), lambda i: (0, 0)),
        ],
        out_specs=pl.BlockSpec((BLK, HID), lambda i: (i, 0)),
        out_shape=jax.ShapeDtypeStruct((N, HID), jnp.float32),
    )(x, emb_blk, W1)


def _scale_body(u_ref, pd_ref, o_ref, d_out_ref):
    d = lax.rsqrt(pd_ref[0, :, 0:1] + pd_ref[1, :, 0:1] - 1.0)
    o_ref[...] = u_ref[...] * d
    d_out_ref[...] = d


def _tc_scale(u1, pdeg):
    return pl.pallas_call(
        _scale_body,
        grid=(N // BLK,),
        in_specs=[
            pl.BlockSpec((BLK, HID), lambda i: (i, 0)),
            pl.BlockSpec((2, BLK, DEG_LANES), lambda i: (0, i, 0)),
        ],
        out_specs=[pl.BlockSpec((BLK, HID), lambda i: (i, 0)),
                   pl.BlockSpec((BLK, 1), lambda i: (i, 0))],
        out_shape=[jax.ShapeDtypeStruct((N, HID), jnp.float32),
                   jax.ShapeDtypeStruct((N, 1), jnp.float32)],
    )(u1, pdeg)


def _mid_body(p_ref, hs_ref, d_ref, b_ref, w_ref, o_ref):
    d = d_ref[...]
    t = (p_ref[0] + p_ref[1] - hs_ref[...]) * d + b_ref[...]
    t = jnp.maximum(t, 0.0)
    o_ref[...] = lax.dot_general(
        t, w_ref[...], (((1,), (0,)), ((), ())),
        preferred_element_type=jnp.float32) * d


def _tc_mid(p, hs, dinv, b, Wn):
    return pl.pallas_call(
        _mid_body,
        grid=(N // BLK,),
        in_specs=[
            pl.BlockSpec((2, BLK, HID), lambda i: (0, i, 0)),
            pl.BlockSpec((BLK, HID), lambda i: (i, 0)),
            pl.BlockSpec((BLK, 1), lambda i: (i, 0)),
            pl.BlockSpec((1, HID), lambda i: (0, 0)),
            pl.BlockSpec((HID, HID), lambda i: (0, 0)),
        ],
        out_specs=pl.BlockSpec((BLK, HID), lambda i: (i, 0)),
        out_shape=jax.ShapeDtypeStruct((N, HID), jnp.float32),
    )(p, hs, dinv, b.reshape(1, HID), Wn)


def _fin_body(p_ref, hs_ref, d_ref, b_ref, pool_ref, wl_ref, bl_ref, o_ref):
    t = (p_ref[0] + p_ref[1] - hs_ref[...]) * d_ref[...] + b_ref[...]
    pooled = lax.dot_general(
        pool_ref[...], t, (((1,), (0,)), ((), ())),
        preferred_element_type=jnp.float32)
    o_ref[0] = jnp.tanh(lax.dot_general(
        pooled, wl_ref[...], (((1,), (0,)), ((), ())),
        preferred_element_type=jnp.float32) + bl_ref[...])


def _tc_fin(p, hs, dinv, b, pool_mat, Wlin, blin, n_graphs, act):
    gblk = BLK // 10
    return pl.pallas_call(
        _fin_body,
        grid=(N // BLK,),
        in_specs=[
            pl.BlockSpec((2, BLK, HID), lambda i: (0, i, 0)),
            pl.BlockSpec((BLK, HID), lambda i: (i, 0)),
            pl.BlockSpec((BLK, 1), lambda i: (i, 0)),
            pl.BlockSpec((1, HID), lambda i: (0, 0)),
            pl.BlockSpec((gblk, BLK), lambda i: (0, 0)),
            pl.BlockSpec((HID, act), lambda i: (0, 0)),
            pl.BlockSpec((1, act), lambda i: (0, 0)),
        ],
        out_specs=pl.BlockSpec((1, gblk, act), lambda i: (i, 0, 0)),
        out_shape=jax.ShapeDtypeStruct((n_graphs // gblk, gblk, act),
                                       jnp.float32),
    )(p, hs, dinv, b.reshape(1, HID), pool_mat, Wlin,
      blin.reshape(1, act)).reshape(n_graphs, act)


def kernel(x, edge_index, batch, node_type_table, W1, b1, W2, b2, W3, b3,
           Wlin, blin):
    n_graphs = batch.shape[0] // 10
    act = Wlin.shape[1]

    src1d = edge_index[0]
    dst3d = edge_index[1].reshape(NW, CHUNKS_PER_W, CHUNK)
    ones = jnp.ones((N, DEG_LANES), jnp.float32)
    ones_c = jnp.ones((CHUNK, DEG_LANES), jnp.float32)

    node_types = jnp.array([0, 0, 0, 0, 0, 0, 0, 1, 2, 3], jnp.int32)
    emb_blk = jnp.tile(node_type_table[node_types], (BLK // 10, 1))

    gblk = BLK // 10
    pool_mat = (jnp.repeat(jnp.arange(gblk, dtype=jnp.int32), 10)[None, :]
                == jnp.arange(gblk, dtype=jnp.int32)[:, None]
                ).astype(jnp.float32) * 0.1

    pdeg = _sc_deg(dst3d, ones, ones_c)
    u1 = _tc_u1(x, emb_blk, W1)
    hs1, dinv = _tc_scale(u1, pdeg)
    p1 = _sc_seg(hs1, src1d, dst3d)
    hs2 = _tc_mid(p1, hs1, dinv, b1, W2)
    p2 = _sc_seg(hs2, src1d, dst3d)
    hs3 = _tc_mid(p2, hs2, dinv, b2, W3)
    p3 = _sc_seg(hs3, src1d, dst3d)
    return _tc_fin(p3, hs3, dinv, b3, pool_mat, Wlin, blin, n_graphs, act)


# speedup vs baseline: 1.0120x; 1.0120x over previous
import functools

import jax
import jax.numpy as jnp
from jax import lax
from jax.experimental import pallas as pl
from jax.experimental.pallas import tpu as pltpu
from jax.experimental.pallas import tpu_sc as plsc

N = 10000
E = 320000
HID = 128
CHUNK = 80
NW = 32
EDGES_PER_W = E // NW
CHUNKS_PER_W = EDGES_PER_W // CHUNK
DEG_LANES = 16
ROWS_A = 640
ROWS_B = N - 15 * ROWS_A

_sc_mesh = functools.partial(
    plsc.VectorSubcoreMesh, core_axis_name="c", subcore_axis_name="s")


def _init_rows(src_hbm_like, acc, s):
    @pl.when(s < 15)
    def _():
        pltpu.sync_copy(src_hbm_like.at[pl.ds(s * ROWS_A, ROWS_A)],
                        acc.at[pl.ds(s * ROWS_A, ROWS_A)])

    @pl.when(s == 15)
    def _():
        pltpu.sync_copy(src_hbm_like.at[pl.ds(15 * ROWS_A, ROWS_B)],
                        acc.at[pl.ds(15 * ROWS_A, ROWS_B)])


def _drain_rows(acc, out_hbm, c, s):
    @pl.when(s < 15)
    def _():
        pltpu.sync_copy(acc.at[pl.ds(s * ROWS_A, ROWS_A)],
                        out_hbm.at[c, pl.ds(s * ROWS_A, ROWS_A)])

    @pl.when(s == 15)
    def _():
        pltpu.sync_copy(acc.at[pl.ds(15 * ROWS_A, ROWS_B)],
                        out_hbm.at[c, pl.ds(15 * ROWS_A, ROWS_B)])


def _deg_kernel(dst_hbm, ones_hbm, ones_c_hbm, out_hbm, dst_slab, ones_v, acc,
                sem):
    c = lax.axis_index("c")
    s = lax.axis_index("s")
    w = c * 16 + s
    _init_rows(ones_hbm, acc, s)
    pltpu.sync_copy(ones_c_hbm, ones_v)
    pltpu.sync_copy(dst_hbm.at[w], dst_slab)
    plsc.subcore_barrier()

    def body(k, carry):
        for b in range(5):
            pltpu.async_copy(ones_v, acc.at[dst_slab.at[5 * k + b]], sem,
                             add=True)
        for b in range(5):
            pltpu.make_async_copy(ones_v, acc.at[dst_slab.at[0]], sem).wait()
        return carry

    lax.fori_loop(0, CHUNKS_PER_W // 5, body, 0)
    plsc.subcore_barrier()
    _drain_rows(acc, out_hbm, c, s)


def _seg_kernel(hs_hbm, src_hbm, dst_hbm, out_hbm, src_v, dst_slab,
                rows_a, rows_b, acc, gsem_a, gsem_b):
    c = lax.axis_index("c")
    s = lax.axis_index("s")
    w = c * 16 + s
    pltpu.sync_copy(src_hbm.at[pl.ds(w * EDGES_PER_W, EDGES_PER_W)], src_v)

    def gather(j, buf, sem):
        return pltpu.async_copy(
            hs_hbm.at[src_v.at[pl.ds(j * CHUNK, CHUNK)]], buf, sem)

    gather(0, rows_a, gsem_a)
    gather(1, rows_b, gsem_b)
    _init_rows(hs_hbm, acc, s)
    pltpu.sync_copy(dst_hbm.at[w], dst_slab)
    plsc.subcore_barrier()

    def drain(buf, sem):
        pltpu.make_async_copy(
            hs_hbm.at[src_v.at[pl.ds(0, CHUNK)]], buf, sem).wait()

    def body(k, carry):
        drain(rows_a, gsem_a)
        pltpu.sync_copy(rows_a, acc.at[dst_slab.at[2 * k]], add=True)
        gather(2 * k + 2, rows_a, gsem_a)
        drain(rows_b, gsem_b)
        pltpu.sync_copy(rows_b, acc.at[dst_slab.at[2 * k + 1]], add=True)
        gather(2 * k + 3, rows_b, gsem_b)
        return carry

    lax.fori_loop(0, (CHUNKS_PER_W - 3) // 2, body, 0)
    drain(rows_a, gsem_a)
    pltpu.sync_copy(rows_a, acc.at[dst_slab.at[CHUNKS_PER_W - 3]], add=True)
    gather(CHUNKS_PER_W - 1, rows_a, gsem_a)
    drain(rows_b, gsem_b)
    pltpu.sync_copy(rows_b, acc.at[dst_slab.at[CHUNKS_PER_W - 2]], add=True)
    drain(rows_a, gsem_a)
    pltpu.sync_copy(rows_a, acc.at[dst_slab.at[CHUNKS_PER_W - 1]], add=True)
    plsc.subcore_barrier()
    _drain_rows(acc, out_hbm, c, s)


def _sc_deg(dst3d, ones, ones_c):
    k = pl.kernel(
        _deg_kernel,
        out_type=jax.ShapeDtypeStruct((2, N, DEG_LANES), jnp.float32),
        mesh=_sc_mesh(),
        scratch_types=[
            pltpu.VMEM((CHUNKS_PER_W, CHUNK), jnp.int32),
            pltpu.VMEM((CHUNK, DEG_LANES), jnp.float32),
            pltpu.VMEM_SHARED((N, DEG_LANES), jnp.float32),
            pltpu.SemaphoreType.DMA,
        ],
    )
    return k(dst3d, ones, ones_c)


def _sc_seg(hs, src1d, dst3d):
    k = pl.kernel(
        _seg_kernel,
        out_type=jax.ShapeDtypeStruct((2, N, HID), jnp.float32),
        mesh=_sc_mesh(),
        scratch_types=[
            pltpu.VMEM((EDGES_PER_W,), jnp.int32),
            pltpu.VMEM((CHUNKS_PER_W, CHUNK), jnp.int32),
            pltpu.VMEM((CHUNK, HID), jnp.float32),
            pltpu.VMEM((CHUNK, HID), jnp.float32),
            pltpu.VMEM_SHARED((N, HID), jnp.float32),
            pltpu.SemaphoreType.DMA,
            pltpu.SemaphoreType.DMA,
        ],
    )
    return k(hs, src1d, dst3d)


BLK = 1000


def _mm1_body(x_ref, e_ref, w_ref, pd_ref, o_ref, d_out_ref):
    d = lax.rsqrt(pd_ref[0, :, 0:1] + pd_ref[1, :, 0:1] - 1.0)
    h = jnp.concatenate([x_ref[...], e_ref[...]], axis=1)
    o_ref[...] = lax.dot_general(
        h, w_ref[...], (((1,), (0,)), ((), ())),
        preferred_element_type=jnp.float32) * d
    d_out_ref[...] = d


def _tc_mm1(x, emb_blk, W1, pdeg):
    return pl.pallas_call(
        _mm1_body,
        grid=(N // BLK,),
        in_specs=[
            pl.BlockSpec((BLK, 124), lambda i: (i, 0)),
            pl.BlockSpec((BLK, 4), lambda i: (0, 0)),
            pl.BlockSpec((HID, HID), lambda i: (0, 0)),
            pl.BlockSpec((2, BLK, DEG_LANES), lambda i: (0, i, 0)),
        ],
        out_specs=[pl.BlockSpec((BLK, HID), lambda i: (i, 0)),
                   pl.BlockSpec((BLK, 1), lambda i: (i, 0))],
        out_shape=[jax.ShapeDtypeStruct((N, HID), jnp.float32),
                   jax.ShapeDtypeStruct((N, 1), jnp.float32)],
    )(x, emb_blk, W1, pdeg)


def _mid_body(p_ref, hs_ref, d_ref, b_ref, w_ref, o_ref):
    d = d_ref[...]
    t = (p_ref[0] + p_ref[1] - hs_ref[...]) * d + b_ref[...]
    t = jnp.maximum(t, 0.0)
    o_ref[...] = lax.dot_general(
        t, w_ref[...], (((1,), (0,)), ((), ())),
        preferred_element_type=jnp.float32) * d


def _tc_mid(p, hs, dinv, b, Wn):
    return pl.pallas_call(
        _mid_body,
        grid=(N // BLK,),
        in_specs=[
            pl.BlockSpec((2, BLK, HID), lambda i: (0, i, 0)),
            pl.BlockSpec((BLK, HID), lambda i: (i, 0)),
            pl.BlockSpec((BLK, 1), lambda i: (i, 0)),
            pl.BlockSpec((1, HID), lambda i: (0, 0)),
            pl.BlockSpec((HID, HID), lambda i: (0, 0)),
        ],
        out_specs=pl.BlockSpec((BLK, HID), lambda i: (i, 0)),
        out_shape=jax.ShapeDtypeStruct((N, HID), jnp.float32),
    )(p, hs, dinv, b.reshape(1, HID), Wn)


def _fin_body(p_ref, hs_ref, d_ref, b_ref, pool_ref, wl_ref, bl_ref, o_ref):
    t = (p_ref[0] + p_ref[1] - hs_ref[...]) * d_ref[...] + b_ref[...]
    pooled = lax.dot_general(
        pool_ref[...], t, (((1,), (0,)), ((), ())),
        preferred_element_type=jnp.float32)
    o_ref[0] = jnp.tanh(lax.dot_general(
        pooled, wl_ref[...], (((1,), (0,)), ((), ())),
        preferred_element_type=jnp.float32) + bl_ref[...])


def _tc_fin(p, hs, dinv, b, pool_mat, Wlin, blin, n_graphs, act):
    gblk = BLK // 10
    return pl.pallas_call(
        _fin_body,
        grid=(N // BLK,),
        in_specs=[
            pl.BlockSpec((2, BLK, HID), lambda i: (0, i, 0)),
            pl.BlockSpec((BLK, HID), lambda i: (i, 0)),
            pl.BlockSpec((BLK, 1), lambda i: (i, 0)),
            pl.BlockSpec((1, HID), lambda i: (0, 0)),
            pl.BlockSpec((gblk, BLK), lambda i: (0, 0)),
            pl.BlockSpec((HID, act), lambda i: (0, 0)),
            pl.BlockSpec((1, act), lambda i: (0, 0)),
        ],
        out_specs=pl.BlockSpec((1, gblk, act), lambda i: (i, 0, 0)),
        out_shape=jax.ShapeDtypeStruct((n_graphs // gblk, gblk, act),
                                       jnp.float32),
    )(p, hs, dinv, b.reshape(1, HID), pool_mat, Wlin,
      blin.reshape(1, act)).reshape(n_graphs, act)


def kernel(x, edge_index, batch, node_type_table, W1, b1, W2, b2, W3, b3,
           Wlin, blin):
    n_graphs = batch.shape[0] // 10
    act = Wlin.shape[1]

    src1d = edge_index[0]
    dst3d = edge_index[1].reshape(NW, CHUNKS_PER_W, CHUNK)
    ones = jnp.ones((N, DEG_LANES), jnp.float32)
    ones_c = jnp.ones((CHUNK, DEG_LANES), jnp.float32)

    node_types = jnp.array([0, 0, 0, 0, 0, 0, 0, 1, 2, 3], jnp.int32)
    emb_blk = jnp.tile(node_type_table[node_types], (BLK // 10, 1))

    gblk = BLK // 10
    pool_mat = (jnp.repeat(jnp.arange(gblk, dtype=jnp.int32), 10)[None, :]
                == jnp.arange(gblk, dtype=jnp.int32)[:, None]
                ).astype(jnp.float32) * 0.1

    pdeg = _sc_deg(dst3d, ones, ones_c)
    hs1, dinv = _tc_mm1(x, emb_blk, W1, pdeg)
    p1 = _sc_seg(hs1, src1d, dst3d)
    hs2 = _tc_mid(p1, hs1, dinv, b1, W2)
    p2 = _sc_seg(hs2, src1d, dst3d)
    hs3 = _tc_mid(p2, hs2, dinv, b2, W3)
    p3 = _sc_seg(hs3, src1d, dst3d)
    return _tc_fin(p3, hs3, dinv, b3, pool_mat, Wlin, blin, n_graphs, act)
